# baseline (speedup 1.0000x reference)
import functools

import jax
import jax.numpy as jnp
from jax import lax
from jax.experimental import pallas as pl
from jax.experimental.pallas import tpu as pltpu
from jax.experimental.pallas import tpu_sc as plsc

D = 64
_NUM_WORKERS = 32


def _sc_gather6(ent_left2, ent_right2, rel_table2,
                idx_s, idx_o, idx_ss, idx_so, idx_obs, idx_samp):
    B = idx_s.shape[0]
    bpw = B // _NUM_WORKERS
    mesh = plsc.VectorSubcoreMesh(core_axis_name="c", subcore_axis_name="s")
    out_t = tuple(jax.ShapeDtypeStruct((B, 2 * D), jnp.float32)
                  for _ in range(6))

    @functools.partial(
        pl.kernel,
        mesh=mesh,
        out_type=out_t,
        scratch_types=[
            pltpu.VMEM((bpw,), jnp.int32),
            pltpu.VMEM((bpw, 2 * D), jnp.float32),
            pltpu.SemaphoreType.DMA,
        ],
        compiler_params=pltpu.CompilerParams(use_tc_tiling_on_sc=True),
    )
    def k(el, er, rt, i_s, i_o, i_ss, i_so, i_ob, i_sp,
          o_s, o_o, o_ss, o_so, o_ob, o_sp, idx_v, sel_v, sem):
        wid = lax.axis_index("s") * 2 + lax.axis_index("c")
        base = wid * bpw
        ops = ((el, i_s, o_s), (er, i_o, o_o), (el, i_ss, o_ss),
               (er, i_so, o_so), (rt, i_ob, o_ob), (rt, i_sp, o_sp))
        for tab, ih, oh in ops:
            pltpu.sync_copy(ih.at[pl.ds(base, bpw)], idx_v)

            def issue(t, _, tab=tab):
                v = idx_v[pl.ds(t * 16, 16)]
                blk = lax.shift_right_logical(v, 1)
                for l in range(16):
                    pltpu.async_copy(tab.at[blk[l]], sel_v.at[t * 16 + l],
                                     sem)
                return _

            lax.fori_loop(0, bpw // 16, issue, 0)

            def drain(i, _, tab=tab):
                pltpu.make_async_copy(tab.at[0], sel_v.at[0], sem).wait()
                return _

            lax.fori_loop(0, bpw, drain, 0)
            pltpu.sync_copy(sel_v, oh.at[pl.ds(base, bpw)])

    return k(ent_left2, ent_right2, rel_table2,
             idx_s, idx_o, idx_ss, idx_so, idx_obs, idx_samp)


def _logsig(x):
    return jnp.minimum(x, 0.0) - jnp.log1p(jnp.exp(-jnp.abs(x)))


def _half(x, h):
    sel = lax.bitwise_and(h, 1)[:, None] == 1
    return jnp.where(sel, x[:, D:], x[:, :D])


def _tc_body(s, o, ss, so, ob, sp, hs, ho, hss, hso, hob, hsp, w1, w2, bias,
             pred_ref, pobs_ref, psamp_ref, loss_ref):
    sv = _half(s[...], hs[...])
    ov = _half(o[...], ho[...])
    ssv = _half(ss[...], hss[...])
    sov = _half(so[...], hso[...])
    obv = _half(ob[...], hob[...])
    spv = _half(sp[...], hsp[...])
    a = jnp.dot(sv, w1[...], preferred_element_type=jnp.float32)
    bo = jnp.dot(ov, w2[...], preferred_element_type=jnp.float32)
    c = jnp.dot(ssv, w1[...], preferred_element_type=jnp.float32)
    e = jnp.dot(sov, w2[...], preferred_element_type=jnp.float32)
    bb = bias[...]
    pred = jnp.tanh(a + bo + bb)
    pss = jnp.tanh(c + bo + bb)
    pso = jnp.tanh(a + e + bb)
    pos = jnp.sum(pred * obv, axis=-1)
    neg = jnp.sum(pred * spv, axis=-1)
    nss = jnp.sum(pss * obv, axis=-1)
    nso = jnp.sum(pso * obv, axis=-1)
    pred_ref[...] = pred
    pobs_ref[...] = jax.nn.sigmoid(pos)
    psamp_ref[...] = jax.nn.sigmoid(neg)
    part = -(jnp.sum(_logsig(pos)) + 2.0 * jnp.sum(_logsig(-neg))
             + 0.5 * jnp.sum(_logsig(-nss)) + 0.5 * jnp.sum(_logsig(-nso)))

    @pl.when(pl.program_id(0) == 0)
    def _():
        loss_ref[...] = jnp.zeros_like(loss_ref)

    loss_ref[...] += jnp.reshape(part, (1, 1))


def _tc_compute(embs, idxs, W1, W2, b2, interpret=False):
    B = embs[0].shape[0]
    BB = 2048
    nb = B // BB
    row = pl.BlockSpec((BB, 2 * D), lambda i: (i, 0))
    full = pl.BlockSpec((D, D), lambda i: (0, 0))
    vec = pl.BlockSpec((BB,), lambda i: (i,))
    return pl.pallas_call(
        _tc_body,
        grid=(nb,),
        in_specs=[row] * 6 + [vec] * 6 + [full, full,
                                          pl.BlockSpec((1, D),
                                                       lambda i: (0, 0))],
        out_specs=[pl.BlockSpec((BB, D), lambda i: (i, 0)), vec, vec,
                   pl.BlockSpec((1, 1), lambda i: (0, 0))],
        out_shape=[
            jax.ShapeDtypeStruct((B, D), jnp.float32),
            jax.ShapeDtypeStruct((B,), jnp.float32),
            jax.ShapeDtypeStruct((B,), jnp.float32),
            jax.ShapeDtypeStruct((1, 1), jnp.float32),
        ],
        interpret=interpret,
    )(*embs, *idxs, W1, W2, b2)


def kernel(subjects, objects, observed_relations, sampled_relations,
           sampled_subjects, sampled_objects,
           ent_left, ent_right, rel_table, W, b):
    idx_s = subjects.astype(jnp.int32)
    idx_o = objects.astype(jnp.int32)
    idx_ss = sampled_subjects.astype(jnp.int32)
    idx_so = sampled_objects.astype(jnp.int32)
    idx_obs = observed_relations[:, 0].astype(jnp.int32)
    idx_samp = sampled_relations[:, 0].astype(jnp.int32)

    el2 = ent_left.reshape(ent_left.shape[0] // 2, 2 * D)
    er2 = ent_right.reshape(ent_right.shape[0] // 2, 2 * D)
    rt2 = rel_table.reshape(rel_table.shape[0] // 2, 2 * D)

    embs = _sc_gather6(el2, er2, rt2,
                       idx_s, idx_o, idx_ss, idx_so, idx_obs, idx_samp)
    idxs = (idx_s, idx_o, idx_ss, idx_so, idx_obs, idx_samp)

    W1 = W[:D]
    W2 = W[D:]
    b2 = b.reshape(1, D)
    pred, pobs, psamp, loss = _tc_compute(embs, idxs, W1, W2, b2)
    return pred, loss[0, 0], pobs, psamp


# speedup vs baseline: 2.2456x; 2.2456x over previous
import functools

import jax
import jax.numpy as jnp
from jax import lax
from jax.experimental import pallas as pl
from jax.experimental.pallas import tpu as pltpu
from jax.experimental.pallas import tpu_sc as plsc

D = 64
_NUM_WORKERS = 32


def _sc_gather6(ent_left3, ent_right3, rel_table3,
                idx_s, idx_o, idx_ss, idx_so, idx_obs, idx_samp):
    B = idx_s.shape[0]
    bpw = B // _NUM_WORKERS
    mesh = plsc.VectorSubcoreMesh(core_axis_name="c", subcore_axis_name="s")
    out_t = tuple(jax.ShapeDtypeStruct((B, D), jnp.float32) for _ in range(6))

    @functools.partial(
        pl.kernel,
        mesh=mesh,
        out_type=out_t,
        scratch_types=[
            pltpu.VMEM((bpw,), jnp.int32),
            pltpu.VMEM((bpw // 2, D), jnp.float32),
            pltpu.VMEM((bpw // 2, D), jnp.float32),
            pltpu.SemaphoreType.DMA,
            pltpu.SemaphoreType.DMA,
        ],
        compiler_params=pltpu.CompilerParams(use_tc_tiling_on_sc=True),
    )
    def k(el, er, rt, i_s, i_o, i_ss, i_so, i_ob, i_sp,
          o_s, o_o, o_ss, o_so, o_ob, o_sp, idx_v, sel0, sel1, sem0, sem1):
        wid = lax.axis_index("s") * 2 + lax.axis_index("c")
        base = wid * bpw
        bpc = bpw // 2
        ops = ((el, i_s, o_s), (er, i_o, o_o), (el, i_ss, o_ss),
               (er, i_so, o_so), (rt, i_ob, o_ob), (rt, i_sp, o_sp))
        bufs = ((sel0, sem0), (sel1, sem1))

        def issue_chunk(tab, ih, half, sel, sem):
            if half == 0:
                pltpu.sync_copy(ih.at[pl.ds(base, bpw)], idx_v)

            def issue(t, _, tab=tab, sel=sel, sem=sem, half=half):
                v = idx_v[pl.ds(half * bpc + t * 16, 16)]
                blk = lax.shift_right_logical(v, 3)
                row = lax.bitwise_and(v, 7)
                for l in range(16):
                    pltpu.async_copy(tab.at[blk[l], row[l]],
                                     sel.at[t * 16 + l], sem)
                return _

            lax.fori_loop(0, bpc // 16, issue, 0)

        def finish_chunk(tab, oh, half, sel, sem):
            def drain(i, _, tab=tab, sel=sel, sem=sem):
                pltpu.make_async_copy(tab.at[0, 0], sel.at[0], sem).wait()
                return _

            lax.fori_loop(0, bpc, drain, 0)
            pltpu.sync_copy(sel, oh.at[pl.ds(base + half * bpc, bpc)])

        stages = [(tab, ih, oh, half)
                  for (tab, ih, oh) in ops for half in (0, 1)]
        prev = None
        for kk, (tab, ih, oh, half) in enumerate(stages):
            sel, sem = bufs[kk % 2]
            issue_chunk(tab, ih, half, sel, sem)
            if prev is not None:
                finish_chunk(*prev)
            prev = (tab, oh, half, sel, sem)
        finish_chunk(*prev)

    return k(ent_left3, ent_right3, rel_table3,
             idx_s, idx_o, idx_ss, idx_so, idx_obs, idx_samp)


def _logsig(x):
    return jnp.minimum(x, 0.0) - jnp.log1p(jnp.exp(-jnp.abs(x)))


def _tc_body(s, o, ss, so, ob, sp, w1, w2, bias,
             pred_ref, pobs_ref, psamp_ref, loss_ref):
    a = jnp.dot(s[...], w1[...], preferred_element_type=jnp.float32)
    bo = jnp.dot(o[...], w2[...], preferred_element_type=jnp.float32)
    c = jnp.dot(ss[...], w1[...], preferred_element_type=jnp.float32)
    e = jnp.dot(so[...], w2[...], preferred_element_type=jnp.float32)
    bb = bias[...]
    pred = jnp.tanh(a + bo + bb)
    pss = jnp.tanh(c + bo + bb)
    pso = jnp.tanh(a + e + bb)
    obv = ob[...]
    spv = sp[...]
    pos = jnp.sum(pred * obv, axis=-1)
    neg = jnp.sum(pred * spv, axis=-1)
    nss = jnp.sum(pss * obv, axis=-1)
    nso = jnp.sum(pso * obv, axis=-1)
    pred_ref[...] = pred.T
    pobs_ref[...] = jax.nn.sigmoid(pos)
    psamp_ref[...] = jax.nn.sigmoid(neg)
    part = -(jnp.sum(_logsig(pos)) + 2.0 * jnp.sum(_logsig(-neg))
             + 0.5 * jnp.sum(_logsig(-nss)) + 0.5 * jnp.sum(_logsig(-nso)))

    @pl.when(pl.program_id(0) == 0)
    def _():
        loss_ref[...] = jnp.zeros_like(loss_ref)

    loss_ref[...] += jnp.reshape(part, (1, 1))


def _tc_compute(s_emb, o_emb, ss_emb, so_emb, obs_emb, samp_emb, W1, W2, b2,
                interpret=False):
    B = s_emb.shape[0]
    BB = 2048
    nb = B // BB
    row = pl.BlockSpec((BB, D), lambda i: (i, 0))
    full = pl.BlockSpec((D, D), lambda i: (0, 0))
    vec = pl.BlockSpec((BB,), lambda i: (i,))
    return pl.pallas_call(
        _tc_body,
        grid=(nb,),
        in_specs=[row, row, row, row, row, row, full, full,
                  pl.BlockSpec((1, D), lambda i: (0, 0))],
        out_specs=[pl.BlockSpec((D, BB), lambda i: (0, i)), vec, vec,
                   pl.BlockSpec((1, 1), lambda i: (0, 0))],
        out_shape=[
            jax.ShapeDtypeStruct((D, B), jnp.float32),
            jax.ShapeDtypeStruct((B,), jnp.float32),
            jax.ShapeDtypeStruct((B,), jnp.float32),
            jax.ShapeDtypeStruct((1, 1), jnp.float32),
        ],
        interpret=interpret,
    )(s_emb, o_emb, ss_emb, so_emb, obs_emb, samp_emb, W1, W2, b2)


def kernel(subjects, objects, observed_relations, sampled_relations,
           sampled_subjects, sampled_objects,
           ent_left, ent_right, rel_table, W, b):
    idx_s = subjects.astype(jnp.int32)
    idx_o = objects.astype(jnp.int32)
    idx_ss = sampled_subjects.astype(jnp.int32)
    idx_so = sampled_objects.astype(jnp.int32)
    idx_obs = observed_relations[:, 0].astype(jnp.int32)
    idx_samp = sampled_relations[:, 0].astype(jnp.int32)

    el3 = ent_left.reshape(ent_left.shape[0] // 8, 8, D)
    er3 = ent_right.reshape(ent_right.shape[0] // 8, 8, D)
    rt3 = rel_table.reshape(rel_table.shape[0] // 8, 8, D)

    s_emb, o_emb, ss_emb, so_emb, obs_emb, samp_emb = _sc_gather6(
        el3, er3, rt3, idx_s, idx_o, idx_ss, idx_so, idx_obs, idx_samp)

    W1 = W[:D]
    W2 = W[D:]
    b2 = b.reshape(1, D)
    pred_t, pobs, psamp, loss = _tc_compute(
        s_emb, o_emb, ss_emb, so_emb, obs_emb, samp_emb, W1, W2, b2)
    return pred_t.T, loss[0, 0], pobs, psamp
